# baseline (speedup 1.0000x reference)
import functools

import numpy as np
import jax
import jax.numpy as jnp
from jax import lax
from jax.experimental import pallas as pl
from jax.experimental.pallas import tpu as pltpu
from jax.experimental.pallas import tpu_sc as plsc

B = 2
NQ = 4096
NKV = 4096
C = 384
NH = 6
HD = C // NH
SR = 2
NS = 1024
SCALE = HD ** -0.5
INV4 = 1.0 / (4.0 + 1e-6)
EPS = 1e-5
N_INIT = 128 * 128
TOT_ROWS = B * N_INIT
CHUNK = 128


def _build_perm():
    kh = np.arange(2).reshape(2, 1, 1, 1, 1, 1)
    kw = np.arange(2).reshape(1, 2, 1, 1, 1, 1)
    a = np.arange(2).reshape(1, 1, 2, 1, 1, 1)
    b = np.arange(2).reshape(1, 1, 1, 2, 1, 1)
    r = np.arange(32).reshape(1, 1, 1, 1, 32, 1)
    c = np.arange(32).reshape(1, 1, 1, 1, 1, 32)
    i = 4 * r + 2 * kh + a
    j = 4 * c + 2 * kw + b
    return jnp.asarray((i * 128 + j).reshape(-1), jnp.int32)


_PERM = _build_perm()


def _sc_gather(table, idx2d):
    CW = C // 2
    info = plsc.get_sparse_core_info()
    nw = info.num_cores * info.num_subcores
    per_w = TOT_ROWS // nw
    nch = per_w // CHUNK
    mesh = plsc.VectorSubcoreMesh(core_axis_name="c", subcore_axis_name="s")

    @functools.partial(
        pl.kernel,
        mesh=mesh,
        out_type=jax.ShapeDtypeStruct((TOT_ROWS, CW), jnp.int32),
        compiler_params=pltpu.CompilerParams(use_tc_tiling_on_sc=False),
        scratch_types=[
            pltpu.VMEM((nch, CHUNK), jnp.int32),
            pltpu.VMEM((CHUNK, CW), jnp.int32),
            pltpu.VMEM((CHUNK, CW), jnp.int32),
            pltpu.SemaphoreType.DMA,
            pltpu.SemaphoreType.DMA,
        ],
    )
    def gk(table_hbm, idx_hbm, out_hbm, idx_v, buf0, buf1, sem0, sem1):
        wid = lax.axis_index("s") * info.num_cores + lax.axis_index("c")
        base = wid * per_w
        pltpu.sync_copy(idx_hbm.at[pl.ds(wid * nch, nch)], idx_v)
        bufs, sems = (buf0, buf1), (sem0, sem1)
        pltpu.async_copy(table_hbm.at[idx_v.at[0]], buf0, sem0)

        def outer(oc, _):
            for t in range(2):
                ci = oc * 2 + t
                nxt = ci + 1

                @pl.when(nxt < nch)
                def _():
                    pltpu.async_copy(table_hbm.at[idx_v.at[nxt]],
                                     bufs[1 - t], sems[1 - t])

                pltpu.make_async_copy(table_hbm.at[idx_v.at[ci]],
                                      bufs[t], sems[t]).wait()
                pltpu.sync_copy(bufs[t],
                                out_hbm.at[pl.ds(base + ci * CHUNK, CHUNK)])
            return 0

        lax.fori_loop(0, nch // 2, outer, 0)

    return gk(table, idx2d)


def _kv_path(g5, w2s, srb, lng, lnb, wkv):
    BCC = 256

    def body(g_ref, w2_ref, srb_ref, lng_ref, lnb_ref, wkv_ref, out_ref):
        acc = jnp.broadcast_to(srb_ref[...], (BCC, C)).astype(jnp.float32)
        for k in range(4):
            mk = (g_ref[0, k, 0].astype(jnp.float32)
                  + g_ref[0, k, 1].astype(jnp.float32)
                  + g_ref[0, k, 2].astype(jnp.float32)
                  + g_ref[0, k, 3].astype(jnp.float32))
            acc = acc + jnp.dot(mk.astype(jnp.bfloat16), w2_ref[k],
                                preferred_element_type=jnp.float32)
        mu = jnp.mean(acc, axis=-1, keepdims=True)
        xc = acc - mu
        var = jnp.mean(xc * xc, axis=-1, keepdims=True)
        ln = xc * lax.rsqrt(var + EPS) * lng_ref[...] + lnb_ref[...]
        out_ref[0] = jnp.dot(ln.astype(jnp.bfloat16), wkv_ref[...],
                             preferred_element_type=jnp.float32
                             ).astype(jnp.bfloat16)

    return pl.pallas_call(
        body,
        grid=(B, NS // BCC),
        in_specs=[
            pl.BlockSpec((1, 4, 4, BCC, C), lambda b, i: (b, 0, 0, i, 0)),
            pl.BlockSpec((4, C, C), lambda b, i: (0, 0, 0)),
            pl.BlockSpec((1, C), lambda b, i: (0, 0)),
            pl.BlockSpec((1, C), lambda b, i: (0, 0)),
            pl.BlockSpec((1, C), lambda b, i: (0, 0)),
            pl.BlockSpec((C, 2 * C), lambda b, i: (0, 0)),
        ],
        out_specs=pl.BlockSpec((1, BCC, 2 * C), lambda b, i: (b, i, 0)),
        out_shape=jax.ShapeDtypeStruct((B, NS, 2 * C), jnp.bfloat16),
    )(g5, w2s, srb, lng, lnb, wkv)


def _attention(q_x, wqs, kv, wp, bp):
    BQ = 512

    def body(qx_ref, wq_ref, kv_ref, wp_ref, bp_ref, out_ref):
        q = jnp.dot(qx_ref[0], wq_ref[...],
                    preferred_element_type=jnp.float32).astype(jnp.bfloat16)
        outs = []
        for h in range(NH):
            qh = q[:, h * HD:(h + 1) * HD]
            kh = kv_ref[0][:, h * HD:(h + 1) * HD]
            vh = kv_ref[0][:, C + h * HD:C + (h + 1) * HD]
            s = lax.dot_general(qh, kh, (((1,), (1,)), ((), ())),
                                preferred_element_type=jnp.float32)
            m = jnp.max(s, axis=-1, keepdims=True)
            p = jnp.exp(s - m).astype(jnp.bfloat16)
            d = jnp.sum(p, axis=-1, keepdims=True,
                        dtype=jnp.float32)
            outs.append(jnp.dot(p, vh, preferred_element_type=jnp.float32) / d)
        acc = jnp.concatenate(outs, axis=-1)
        out_ref[0] = jnp.dot(acc.astype(jnp.bfloat16), wp_ref[...],
                             preferred_element_type=jnp.float32) + bp_ref[...]

    return pl.pallas_call(
        body,
        grid=(B, NQ // BQ),
        in_specs=[
            pl.BlockSpec((1, BQ, C), lambda b, i: (b, i, 0)),
            pl.BlockSpec((C, C), lambda b, i: (0, 0)),
            pl.BlockSpec((1, NS, 2 * C), lambda b, i: (b, 0, 0)),
            pl.BlockSpec((C, C), lambda b, i: (0, 0)),
            pl.BlockSpec((1, C), lambda b, i: (0, 0)),
        ],
        out_specs=pl.BlockSpec((1, BQ, C), lambda b, i: (b, i, 0)),
        out_shape=jax.ShapeDtypeStruct((B, NQ, C), jnp.float32),
    )(q_x, wqs, kv, wp, bp)


def kernel(q_x, kv_x, idx_token, Wq, Wkv, sr_w, sr_b, ln_g, ln_b, Wp, bp):
    idx32 = idx_token.astype(jnp.int32)
    flat_idx = (idx32[:, _PERM]
                + (jnp.arange(B, dtype=jnp.int32) * NKV)[:, None])
    idx2d = flat_idx.reshape(TOT_ROWS // CHUNK, CHUNK)
    table = lax.bitcast_convert_type(
        kv_x.astype(jnp.bfloat16).reshape(B * NKV, C // 2, 2), jnp.int32)
    g = _sc_gather(table, idx2d)
    g5 = lax.bitcast_convert_type(g, jnp.bfloat16).reshape(B, 4, 4, NS, C)
    w2s = (jnp.transpose(sr_w, (2, 3, 1, 0)).reshape(4, C, C)
           * INV4).astype(jnp.bfloat16)
    kv = _kv_path(g5, w2s, sr_b.reshape(1, C), ln_g.reshape(1, C),
                  ln_b.reshape(1, C), Wkv.astype(jnp.bfloat16))
    return _attention(q_x.astype(jnp.bfloat16),
                      (Wq * SCALE).astype(jnp.bfloat16), kv,
                      Wp.astype(jnp.bfloat16), bp.reshape(1, C))


# speedup vs baseline: 2.4786x; 2.4786x over previous
import functools

import numpy as np
import jax
import jax.numpy as jnp
from jax import lax
from jax.experimental import pallas as pl
from jax.experimental.pallas import tpu as pltpu
from jax.experimental.pallas import tpu_sc as plsc

B = 2
NQ = 4096
NKV = 4096
C = 384
NH = 6
HD = C // NH
SR = 2
NS = 1024
SCALE = HD ** -0.5
INV4 = 1.0 / (4.0 + 1e-6)
EPS = 1e-5
N_INIT = 128 * 128
TOT_ROWS = B * N_INIT
CHUNK = 128


def _build_perm():
    kh = np.arange(2).reshape(2, 1, 1, 1, 1, 1)
    kw = np.arange(2).reshape(1, 2, 1, 1, 1, 1)
    a = np.arange(2).reshape(1, 1, 2, 1, 1, 1)
    b = np.arange(2).reshape(1, 1, 1, 2, 1, 1)
    r = np.arange(32).reshape(1, 1, 1, 1, 32, 1)
    c = np.arange(32).reshape(1, 1, 1, 1, 1, 32)
    i = 4 * r + 2 * kh + a
    j = 4 * c + 2 * kw + b
    return jnp.asarray((i * 128 + j).reshape(-1), jnp.int32)


_PERM = _build_perm()


def _sc_gather(table, idx2d):
    CW = C
    info = plsc.get_sparse_core_info()
    nw = info.num_cores * info.num_subcores
    per_w = TOT_ROWS // nw
    nch = per_w // CHUNK
    mesh = plsc.VectorSubcoreMesh(core_axis_name="c", subcore_axis_name="s")

    @functools.partial(
        pl.kernel,
        mesh=mesh,
        out_type=jax.ShapeDtypeStruct((TOT_ROWS, CW), jnp.float32),
        scratch_types=[
            pltpu.VMEM((nch, CHUNK), jnp.int32),
            pltpu.VMEM((CHUNK, CW), jnp.float32),
            pltpu.VMEM((CHUNK, CW), jnp.float32),
            pltpu.SemaphoreType.DMA,
            pltpu.SemaphoreType.DMA,
        ],
    )
    def gk(table_hbm, idx_hbm, out_hbm, idx_v, buf0, buf1, sem0, sem1):
        wid = lax.axis_index("s") * info.num_cores + lax.axis_index("c")
        base = wid * per_w
        pltpu.sync_copy(idx_hbm.at[pl.ds(wid * nch, nch)], idx_v)
        bufs, sems = (buf0, buf1), (sem0, sem1)
        pltpu.async_copy(table_hbm.at[idx_v.at[0]], buf0, sem0)

        def outer(oc, _):
            for t in range(2):
                ci = oc * 2 + t
                nxt = ci + 1

                @pl.when(nxt < nch)
                def _():
                    pltpu.async_copy(table_hbm.at[idx_v.at[nxt]],
                                     bufs[1 - t], sems[1 - t])

                pltpu.make_async_copy(table_hbm.at[idx_v.at[ci]],
                                      bufs[t], sems[t]).wait()
                pltpu.sync_copy(bufs[t],
                                out_hbm.at[pl.ds(base + ci * CHUNK, CHUNK)])
            return 0

        lax.fori_loop(0, nch // 2, outer, 0)

    return gk(table, idx2d)


def _kv_path(g5, w2s, srb, lng, lnb, wkv):
    BCC = 256

    def body(g_ref, w2_ref, srb_ref, lng_ref, lnb_ref, wkv_ref, out_ref):
        acc = jnp.broadcast_to(srb_ref[...], (BCC, C)).astype(jnp.float32)
        for k in range(4):
            mk = (g_ref[0, k, 0].astype(jnp.float32)
                  + g_ref[0, k, 1].astype(jnp.float32)
                  + g_ref[0, k, 2].astype(jnp.float32)
                  + g_ref[0, k, 3].astype(jnp.float32))
            acc = acc + jnp.dot(mk.astype(jnp.bfloat16), w2_ref[k],
                                preferred_element_type=jnp.float32)
        mu = jnp.mean(acc, axis=-1, keepdims=True)
        xc = acc - mu
        var = jnp.mean(xc * xc, axis=-1, keepdims=True)
        ln = xc * lax.rsqrt(var + EPS) * lng_ref[...] + lnb_ref[...]
        out_ref[0] = jnp.dot(ln.astype(jnp.bfloat16), wkv_ref[...],
                             preferred_element_type=jnp.float32
                             ).astype(jnp.bfloat16)

    return pl.pallas_call(
        body,
        grid=(B, NS // BCC),
        in_specs=[
            pl.BlockSpec((1, 4, 4, BCC, C), lambda b, i: (b, 0, 0, i, 0)),
            pl.BlockSpec((4, C, C), lambda b, i: (0, 0, 0)),
            pl.BlockSpec((1, C), lambda b, i: (0, 0)),
            pl.BlockSpec((1, C), lambda b, i: (0, 0)),
            pl.BlockSpec((1, C), lambda b, i: (0, 0)),
            pl.BlockSpec((C, 2 * C), lambda b, i: (0, 0)),
        ],
        out_specs=pl.BlockSpec((1, BCC, 2 * C), lambda b, i: (b, i, 0)),
        out_shape=jax.ShapeDtypeStruct((B, NS, 2 * C), jnp.bfloat16),
    )(g5, w2s, srb, lng, lnb, wkv)


def _attention(q_x, wqs, kv, wp, bp):
    BQ = 512

    def body(qx_ref, wq_ref, kv_ref, wp_ref, bp_ref, out_ref):
        q = jnp.dot(qx_ref[0], wq_ref[...],
                    preferred_element_type=jnp.float32).astype(jnp.bfloat16)
        outs = []
        for h in range(NH):
            qh = q[:, h * HD:(h + 1) * HD]
            kh = kv_ref[0][:, h * HD:(h + 1) * HD]
            vh = kv_ref[0][:, C + h * HD:C + (h + 1) * HD]
            s = lax.dot_general(qh, kh, (((1,), (1,)), ((), ())),
                                preferred_element_type=jnp.float32)
            m = jnp.max(s, axis=-1, keepdims=True)
            p = jnp.exp(s - m).astype(jnp.bfloat16)
            d = jnp.sum(p, axis=-1, keepdims=True,
                        dtype=jnp.float32)
            outs.append(jnp.dot(p, vh, preferred_element_type=jnp.float32) / d)
        acc = jnp.concatenate(outs, axis=-1)
        out_ref[0] = jnp.dot(acc.astype(jnp.bfloat16), wp_ref[...],
                             preferred_element_type=jnp.float32) + bp_ref[...]

    return pl.pallas_call(
        body,
        grid=(B, NQ // BQ),
        in_specs=[
            pl.BlockSpec((1, BQ, C), lambda b, i: (b, i, 0)),
            pl.BlockSpec((C, C), lambda b, i: (0, 0)),
            pl.BlockSpec((1, NS, 2 * C), lambda b, i: (b, 0, 0)),
            pl.BlockSpec((C, C), lambda b, i: (0, 0)),
            pl.BlockSpec((1, C), lambda b, i: (0, 0)),
        ],
        out_specs=pl.BlockSpec((1, BQ, C), lambda b, i: (b, i, 0)),
        out_shape=jax.ShapeDtypeStruct((B, NQ, C), jnp.float32),
    )(q_x, wqs, kv, wp, bp)


def kernel(q_x, kv_x, idx_token, Wq, Wkv, sr_w, sr_b, ln_g, ln_b, Wp, bp):
    idx32 = idx_token.astype(jnp.int32)
    flat_idx = (idx32[:, _PERM]
                + (jnp.arange(B, dtype=jnp.int32) * NKV)[:, None])
    idx2d = flat_idx.reshape(TOT_ROWS // CHUNK, CHUNK)
    table = kv_x.reshape(B * NKV, C)
    g = _sc_gather(table, idx2d)
    g5 = g.reshape(B, 4, 4, NS, C)
    w2s = (jnp.transpose(sr_w, (2, 3, 1, 0)).reshape(4, C, C)
           * INV4).astype(jnp.bfloat16)
    kv = _kv_path(g5, w2s, sr_b.reshape(1, C), ln_g.reshape(1, C),
                  ln_b.reshape(1, C), Wkv.astype(jnp.bfloat16))
    return _attention(q_x.astype(jnp.bfloat16),
                      (Wq * SCALE).astype(jnp.bfloat16), kv,
                      Wp.astype(jnp.bfloat16), bp.reshape(1, C))
